# baseline (speedup 1.0000x reference)
import dataclasses
import functools

import jax
import jax.numpy as jnp
from jax import lax
from jax.experimental import pallas as pl
from jax.experimental.pallas import tpu as pltpu
from jax.experimental.pallas import tpu_sc as plsc

_N_ITEMS = 10000
_N_PERSONS = 100000
_N_GRADES = 5
_IP = 10240
_PP = 100352
_TPH = _PP // 2
_NW = 32
_N_RESP = 1000000
_CH = 800
_NCHUNKS = _N_RESP // _CH
_BIG = 30000.0
_CLAMP = 30.0
_LOG2PI = 1.8378770664093453
_LN2 = 0.6931471805599453


def _bf16_bits(x):
    u = lax.bitcast_convert_type(x, jnp.int32)
    return ((u + 0x7FFF + ((u >> 16) & 1)) >> 16) & 0xFFFF


def _pack_pair(lo, hi):
    return (_bf16_bits(hi) << 16) | _bf16_bits(lo)


def _prep_body(a_ref, bb_ref, bd_ref, t_ref, c_ref, a_out_ref, tpk_ref,
               prior_ref):
    rows = lax.broadcasted_iota(jnp.int32, (80, 128), 0)
    cols = lax.broadcasted_iota(jnp.int32, (80, 128), 1)
    item_idx = rows * 128 + cols
    valid_item = item_idx < _N_ITEMS

    a_raw = a_ref[...]
    a = jnp.log(1.0 + jnp.exp(a_raw))
    g1 = jnp.log(1.0 + jnp.exp(bd_ref[0]))
    g2 = jnp.log(1.0 + jnp.exp(bd_ref[1]))
    g3 = jnp.log(1.0 + jnp.exp(bd_ref[2]))
    b1 = bb_ref[...]
    b2 = b1 + g1
    b3 = b2 + g2
    b4 = b3 + g3

    def npdf_sum(x, mask):
        return jnp.sum(jnp.where(mask, -0.5 * x * x - 0.5 * _LOG2PI, 0.0))

    t_all = t_ref[...]
    prows = lax.broadcasted_iota(jnp.int32, (784, 128), 0)
    pcols = lax.broadcasted_iota(jnp.int32, (784, 128), 1)
    valid_person = (prows * 128 + pcols) < _N_PERSONS
    prior = (npdf_sum(a, valid_item)
             + npdf_sum(b1, valid_item) + npdf_sum(b2, valid_item)
             + npdf_sum(b3, valid_item) + npdf_sum(b4, valid_item)
             + npdf_sum(t_all, valid_person))
    prior_ref[...] = jnp.full((1, 1), prior, jnp.float32)

    a_out_ref[...] = jnp.where(valid_item, a, 1.0)
    ab_raw = (a * -1000.0, a * b1, a * b2, a * b3, a * b4, a * 1000.0)
    pad_val = (-_BIG, _BIG, _BIG, _BIG, _BIG, _BIG)
    ab = tuple(jnp.where(valid_item, ab_raw[s], pad_val[s]) for s in range(6))
    for w in range(3):
        c_ref[w] = _pack_pair(ab[2 * w], ab[2 * w + 1])
    tpk_ref[...] = _pack_pair(t_ref[0:392], t_ref[392:784])


_prep = pl.pallas_call(
    _prep_body,
    out_shape=(
        jax.ShapeDtypeStruct((3, 80, 128), jnp.int32),
        jax.ShapeDtypeStruct((80, 128), jnp.float32),
        jax.ShapeDtypeStruct((392, 128), jnp.int32),
        jax.ShapeDtypeStruct((1, 1), jnp.float32),
    ),
)


@functools.cache
def _get_sc_loglik():
    mesh = plsc.VectorSubcoreMesh(core_axis_name="c", subcore_axis_name="s")
    cp = pltpu.CompilerParams()
    if "needs_layout_passes" in pltpu.CompilerParams.__dataclass_fields__:
        cp = dataclasses.replace(cp, needs_layout_passes=False)
    return pl.kernel(
        _sc_loglik_body,
        out_type=jax.ShapeDtypeStruct((_NW, 16), jnp.float32),
        mesh=mesh,
        scratch_types=[
            pltpu.VMEM((3, _IP), jnp.int32),
            pltpu.VMEM((_IP,), jnp.float32),
            pltpu.VMEM((_TPH,), jnp.int32),
            pltpu.VMEM((16,), jnp.float32),
        ],
        compiler_params=cp,
    )


def _sc_loglik_body(c_hbm, a_hbm, tp_hbm, idx_hbm, out_hbm, c_v, a_v, tp_v,
                    acc_v):
    pltpu.sync_copy(c_hbm, c_v)
    pltpu.sync_copy(a_hbm, a_v)
    pltpu.sync_copy(tp_hbm, tp_v)
    acc_v[...] = jnp.zeros((16,), jnp.float32)
    iota16 = lax.iota(jnp.int32, 16)

    def chunk_body(idx_vmem):
        @pl.loop(0, _CH, step=16)
        def _(r0):
            base = r0 * 3 + iota16 * 3
            item = plsc.load_gather(idx_vmem, [base])
            person = plsc.load_gather(idx_vmem, [base + 1])
            resp = plsc.load_gather(idx_vmem, [base + 2])

            g = resp - 1
            cw1 = plsc.load_gather(c_v, [g >> 1, item])
            cw2 = plsc.load_gather(c_v, [(g + 1) >> 1, item])
            a = plsc.load_gather(a_v, [item])
            in_lo = person < _TPH
            tw = plsc.load_gather(
                tp_v, [jnp.where(in_lo, person, person - _TPH)])
            t_bits = jnp.where(in_lo, tw << 16, tw & jnp.int32(-65536))
            t = plsc.bitcast(t_bits, jnp.float32)
            godd = (g & 1) == 1
            ab_lo = plsc.bitcast(
                jnp.where(godd, cw1 & jnp.int32(-65536), cw1 << 16),
                jnp.float32)
            ab_hi = plsc.bitcast(
                jnp.where(godd, cw2 << 16, cw2 & jnp.int32(-65536)),
                jnp.float32)

            at = a * t
            x1 = jnp.minimum(jnp.maximum(at - ab_lo, -_CLAMP), _CLAMP)
            x2 = jnp.minimum(jnp.maximum(at - ab_hi, -_CLAMP), _CLAMP)
            u = jnp.exp(-x1)
            v = jnp.exp(-x2)
            p = (v - u) / ((1.0 + u) * (1.0 + v))

            bits = plsc.bitcast(p, jnp.int32)
            e = (bits >> 23) - 127
            m = plsc.bitcast((bits & 0x007FFFFF) | 0x3F800000, jnp.float32)
            big = m > 1.4142135
            m = jnp.where(big, 0.5 * m, m)
            ef = (e + big.astype(jnp.int32)).astype(jnp.float32)
            r = (m - 1.0) / (m + 1.0)
            s = r * r
            lm = 2.0 * r * (1.0 + s * (1.0 / 3.0 + s * (0.2 + s * (1.0 / 7.0))))
            acc_v[...] = acc_v[...] + (ef * _LN2 + lm)

    pltpu.emit_pipeline(
        chunk_body,
        grid=(_NCHUNKS,),
        in_specs=[pl.BlockSpec((_CH * 3,), lambda i: (i,))],
        core_axis_name=("c", "s"),
        dimension_semantics=(pltpu.PARALLEL,),
    )(idx_hbm)

    wid = lax.axis_index("c") * 16 + lax.axis_index("s")
    pltpu.sync_copy(acc_v, out_hbm.at[wid])


def kernel(indices, a_, b_base_, b_diff_, t):
    n_resp = indices.shape[0]
    idx_flat = indices.reshape(-1)

    a_in = jnp.pad(a_, (0, _IP - _N_ITEMS)).reshape(80, 128)
    bb_in = jnp.pad(b_base_[:, 0], (0, _IP - _N_ITEMS)).reshape(80, 128)
    bd_in = jnp.pad(b_diff_, ((0, _IP - _N_ITEMS), (0, 0))).T.reshape(3, 80, 128)
    t_in = jnp.pad(t, (0, _PP - _N_PERSONS)).reshape(784, 128)

    c_tab, a_tab, tp_tab, prior = _prep(a_in, bb_in, bd_in, t_in)
    parts = _get_sc_loglik()(
        c_tab.reshape(3, _IP),
        a_tab.reshape(_IP),
        tp_tab.reshape(_TPH),
        idx_flat,
    )
    prior_scale = n_resp / 1000000.0
    return -(jnp.sum(parts) + prior[0, 0] * prior_scale)


# speedup vs baseline: 1.6599x; 1.6599x over previous
import dataclasses
import functools

import jax
import jax.numpy as jnp
from jax import lax
from jax.experimental import pallas as pl
from jax.experimental.pallas import tpu as pltpu
from jax.experimental.pallas import tpu_sc as plsc

_N_ITEMS = 10000
_N_PERSONS = 100000
_N_GRADES = 5
_IP = 10240
_PP = 100352
_TPH = _PP // 2
_NW = 32
_N_RESP = 1000000
_CH = 800
_NCHUNKS = _N_RESP // _CH
_BIG = 30000.0
_CLAMP = 30.0
_LOG2PI = 1.8378770664093453
_LN2 = 0.6931471805599453


def _bf16_bits(x):
    u = lax.bitcast_convert_type(x, jnp.int32)
    return ((u + 0x7FFF + ((u >> 16) & 1)) >> 16) & 0xFFFF


def _pack_pair(lo, hi):
    return (_bf16_bits(hi) << 16) | _bf16_bits(lo)


def _prep_body(a_ref, bb_ref, bd_ref, t_ref, c_ref, a_out_ref, tpk_ref,
               prior_ref):
    rows = lax.broadcasted_iota(jnp.int32, (80, 128), 0)
    cols = lax.broadcasted_iota(jnp.int32, (80, 128), 1)
    item_idx = rows * 128 + cols
    valid_item = item_idx < _N_ITEMS

    a_raw = a_ref[...]
    a = jnp.log(1.0 + jnp.exp(a_raw))
    g1 = jnp.log(1.0 + jnp.exp(bd_ref[0]))
    g2 = jnp.log(1.0 + jnp.exp(bd_ref[1]))
    g3 = jnp.log(1.0 + jnp.exp(bd_ref[2]))
    b1 = bb_ref[...]
    b2 = b1 + g1
    b3 = b2 + g2
    b4 = b3 + g3

    def npdf_sum(x, mask):
        return jnp.sum(jnp.where(mask, -0.5 * x * x - 0.5 * _LOG2PI, 0.0))

    t_all = t_ref[...]
    prows = lax.broadcasted_iota(jnp.int32, (784, 128), 0)
    pcols = lax.broadcasted_iota(jnp.int32, (784, 128), 1)
    valid_person = (prows * 128 + pcols) < _N_PERSONS
    prior = (npdf_sum(a, valid_item)
             + npdf_sum(b1, valid_item) + npdf_sum(b2, valid_item)
             + npdf_sum(b3, valid_item) + npdf_sum(b4, valid_item)
             + npdf_sum(t_all, valid_person))
    prior_ref[...] = jnp.full((1, 1), prior, jnp.float32)

    a_out_ref[...] = jnp.where(valid_item, a, 1.0)
    ab_raw = (a * -1000.0, a * b1, a * b2, a * b3, a * b4, a * 1000.0)
    pad_val = (-_BIG, _BIG, _BIG, _BIG, _BIG, _BIG)
    ab = tuple(jnp.where(valid_item, ab_raw[s], pad_val[s]) for s in range(6))
    for w in range(3):
        c_ref[w] = _pack_pair(ab[2 * w], ab[2 * w + 1])
    tpk_ref[...] = _pack_pair(t_ref[0:392], t_ref[392:784])


_DB = 8192


def _deint_body(idx_ref, item_ref, person_ref, resp_ref):
    x = idx_ref[...]
    lane = lax.broadcasted_iota(jnp.int32, (_DB, 3), 1)
    item_ref[...] = jnp.sum(jnp.where(lane == 0, x, 0), axis=1)
    person_ref[...] = jnp.sum(jnp.where(lane == 1, x, 0), axis=1)
    resp_ref[...] = jnp.sum(jnp.where(lane == 2, x, 0), axis=1)


_deint = pl.pallas_call(
    _deint_body,
    grid=(pl.cdiv(_N_RESP, _DB),),
    in_specs=[pl.BlockSpec((_DB, 3), lambda i: (i, 0))],
    out_specs=(
        pl.BlockSpec((_DB,), lambda i: (i,)),
        pl.BlockSpec((_DB,), lambda i: (i,)),
        pl.BlockSpec((_DB,), lambda i: (i,)),
    ),
    out_shape=(
        jax.ShapeDtypeStruct((_N_RESP,), jnp.int32),
        jax.ShapeDtypeStruct((_N_RESP,), jnp.int32),
        jax.ShapeDtypeStruct((_N_RESP,), jnp.int32),
    ),
)


_prep = pl.pallas_call(
    _prep_body,
    out_shape=(
        jax.ShapeDtypeStruct((3, 80, 128), jnp.int32),
        jax.ShapeDtypeStruct((80, 128), jnp.float32),
        jax.ShapeDtypeStruct((392, 128), jnp.int32),
        jax.ShapeDtypeStruct((1, 1), jnp.float32),
    ),
)


@functools.cache
def _get_sc_loglik():
    mesh = plsc.VectorSubcoreMesh(core_axis_name="c", subcore_axis_name="s")
    cp = pltpu.CompilerParams()
    if "needs_layout_passes" in pltpu.CompilerParams.__dataclass_fields__:
        cp = dataclasses.replace(cp, needs_layout_passes=False)
    return pl.kernel(
        _sc_loglik_body,
        out_type=jax.ShapeDtypeStruct((_NW, 16), jnp.float32),
        mesh=mesh,
        scratch_types=[
            pltpu.VMEM((3, _IP), jnp.int32),
            pltpu.VMEM((_IP,), jnp.float32),
            pltpu.VMEM((_TPH,), jnp.int32),
            pltpu.VMEM((16,), jnp.float32),
        ],
        compiler_params=cp,
    )


def _sc_loglik_body(c_hbm, a_hbm, tp_hbm, item_hbm, person_hbm, resp_hbm,
                    out_hbm, c_v, a_v, tp_v, acc_v):
    pltpu.sync_copy(c_hbm, c_v)
    pltpu.sync_copy(a_hbm, a_v)
    pltpu.sync_copy(tp_hbm, tp_v)
    acc_v[...] = jnp.zeros((16,), jnp.float32)

    def chunk_body(item_vmem, person_vmem, resp_vmem):
        @pl.loop(0, _CH, step=16)
        def _(r0):
            item = item_vmem[pl.ds(r0, 16)]
            person = person_vmem[pl.ds(r0, 16)]
            resp = resp_vmem[pl.ds(r0, 16)]

            g = resp - 1
            cw1 = plsc.load_gather(c_v, [g >> 1, item])
            cw2 = plsc.load_gather(c_v, [(g + 1) >> 1, item])
            a = plsc.load_gather(a_v, [item])
            in_lo = person < _TPH
            tw = plsc.load_gather(
                tp_v, [jnp.where(in_lo, person, person - _TPH)])
            t_bits = jnp.where(in_lo, tw << 16, tw & jnp.int32(-65536))
            t = plsc.bitcast(t_bits, jnp.float32)
            godd = (g & 1) == 1
            ab_lo = plsc.bitcast(
                jnp.where(godd, cw1 & jnp.int32(-65536), cw1 << 16),
                jnp.float32)
            ab_hi = plsc.bitcast(
                jnp.where(godd, cw2 << 16, cw2 & jnp.int32(-65536)),
                jnp.float32)

            at = a * t
            x1 = jnp.minimum(jnp.maximum(at - ab_lo, -_CLAMP), _CLAMP)
            x2 = jnp.minimum(jnp.maximum(at - ab_hi, -_CLAMP), _CLAMP)
            u = jnp.exp(-x1)
            v = jnp.exp(-x2)
            p = (v - u) / ((1.0 + u) * (1.0 + v))

            bits = plsc.bitcast(p, jnp.int32)
            e = (bits >> 23) - 127
            m = plsc.bitcast((bits & 0x007FFFFF) | 0x3F800000, jnp.float32)
            big = m > 1.4142135
            m = jnp.where(big, 0.5 * m, m)
            ef = (e + big.astype(jnp.int32)).astype(jnp.float32)
            r = (m - 1.0) / (m + 1.0)
            s = r * r
            lm = 2.0 * r * (1.0 + s * (1.0 / 3.0 + s * (0.2 + s * (1.0 / 7.0))))
            acc_v[...] = acc_v[...] + (ef * _LN2 + lm)

    pltpu.emit_pipeline(
        chunk_body,
        grid=(_NCHUNKS,),
        in_specs=[pl.BlockSpec((_CH,), lambda i: (i,)),
                  pl.BlockSpec((_CH,), lambda i: (i,)),
                  pl.BlockSpec((_CH,), lambda i: (i,))],
        core_axis_name=("c", "s"),
        dimension_semantics=(pltpu.PARALLEL,),
    )(item_hbm, person_hbm, resp_hbm)

    wid = lax.axis_index("c") * 16 + lax.axis_index("s")
    pltpu.sync_copy(acc_v, out_hbm.at[wid])


def kernel(indices, a_, b_base_, b_diff_, t):
    n_resp = indices.shape[0]
    item_v, person_v, resp_v = _deint(indices)

    a_in = jnp.pad(a_, (0, _IP - _N_ITEMS)).reshape(80, 128)
    bb_in = jnp.pad(b_base_[:, 0], (0, _IP - _N_ITEMS)).reshape(80, 128)
    bd_in = jnp.pad(b_diff_, ((0, _IP - _N_ITEMS), (0, 0))).T.reshape(3, 80, 128)
    t_in = jnp.pad(t, (0, _PP - _N_PERSONS)).reshape(784, 128)

    c_tab, a_tab, tp_tab, prior = _prep(a_in, bb_in, bd_in, t_in)
    parts = _get_sc_loglik()(
        c_tab.reshape(3, _IP),
        a_tab.reshape(_IP),
        tp_tab.reshape(_TPH),
        item_v,
        person_v,
        resp_v,
    )
    prior_scale = n_resp / 1000000.0
    return -(jnp.sum(parts) + prior[0, 0] * prior_scale)


# speedup vs baseline: 15.9172x; 9.5892x over previous
import dataclasses
import functools

import jax
import jax.numpy as jnp
from jax import lax
from jax.experimental import pallas as pl
from jax.experimental.pallas import tpu as pltpu
from jax.experimental.pallas import tpu_sc as plsc

_N_ITEMS = 10000
_N_PERSONS = 100000
_N_GRADES = 5
_IP = 10240
_PP = 100352
_TPH = _PP // 2
_NW = 32
_N_RESP = 1000000
_CH = 800
_NCHUNKS = _N_RESP // _CH
_BIG = 30000.0
_CLAMP = 30.0
_LOG2PI = 1.8378770664093453
_LN2 = 0.6931471805599453


def _bf16_bits(x):
    u = lax.bitcast_convert_type(x, jnp.int32)
    return ((u + 0x7FFF + ((u >> 16) & 1)) >> 16) & 0xFFFF


def _pack_pair(lo, hi):
    return (_bf16_bits(hi) << 16) | _bf16_bits(lo)


def _prep_body(a_ref, bb_ref, bd_ref, t_ref, c_ref, a_out_ref, tpk_ref,
               prior_ref):
    rows = lax.broadcasted_iota(jnp.int32, (80, 128), 0)
    cols = lax.broadcasted_iota(jnp.int32, (80, 128), 1)
    item_idx = rows * 128 + cols
    valid_item = item_idx < _N_ITEMS

    a_raw = a_ref[...]
    a = jnp.log(1.0 + jnp.exp(a_raw))
    g1 = jnp.log(1.0 + jnp.exp(bd_ref[0]))
    g2 = jnp.log(1.0 + jnp.exp(bd_ref[1]))
    g3 = jnp.log(1.0 + jnp.exp(bd_ref[2]))
    b1 = bb_ref[...]
    b2 = b1 + g1
    b3 = b2 + g2
    b4 = b3 + g3

    def npdf_sum(x, mask):
        return jnp.sum(jnp.where(mask, -0.5 * x * x - 0.5 * _LOG2PI, 0.0))

    t_all = t_ref[...]
    prows = lax.broadcasted_iota(jnp.int32, (784, 128), 0)
    pcols = lax.broadcasted_iota(jnp.int32, (784, 128), 1)
    valid_person = (prows * 128 + pcols) < _N_PERSONS
    prior = (npdf_sum(a, valid_item)
             + npdf_sum(b1, valid_item) + npdf_sum(b2, valid_item)
             + npdf_sum(b3, valid_item) + npdf_sum(b4, valid_item)
             + npdf_sum(t_all, valid_person))
    prior_ref[...] = jnp.full((1, 1), prior, jnp.float32)

    a_out_ref[...] = jnp.where(valid_item, a, 1.0)
    ab_raw = (a * -1000.0, a * b1, a * b2, a * b3, a * b4, a * 1000.0)
    pad_val = (-_BIG, _BIG, _BIG, _BIG, _BIG, _BIG)
    ab = tuple(jnp.where(valid_item, ab_raw[s], pad_val[s]) for s in range(6))
    for w in range(3):
        c_ref[w] = _pack_pair(ab[2 * w], ab[2 * w + 1])
    tpk_ref[...] = _pack_pair(t_ref[0:392], t_ref[392:784])


_DB = 8192


def _deint_body(idx_ref, item_ref, person_ref, resp_ref):
    x = idx_ref[...]
    lane = lax.broadcasted_iota(jnp.int32, (_DB, 3), 1)
    item_ref[...] = jnp.sum(jnp.where(lane == 0, x, 0), axis=1)
    person_ref[...] = jnp.sum(jnp.where(lane == 1, x, 0), axis=1)
    resp_ref[...] = jnp.sum(jnp.where(lane == 2, x, 0), axis=1)


_deint = pl.pallas_call(
    _deint_body,
    grid=(pl.cdiv(_N_RESP, _DB),),
    in_specs=[pl.BlockSpec((_DB, 3), lambda i: (i, 0))],
    out_specs=(
        pl.BlockSpec((_DB,), lambda i: (i,)),
        pl.BlockSpec((_DB,), lambda i: (i,)),
        pl.BlockSpec((_DB,), lambda i: (i,)),
    ),
    out_shape=(
        jax.ShapeDtypeStruct((_N_RESP,), jnp.int32),
        jax.ShapeDtypeStruct((_N_RESP,), jnp.int32),
        jax.ShapeDtypeStruct((_N_RESP,), jnp.int32),
    ),
)


_prep = pl.pallas_call(
    _prep_body,
    out_shape=(
        jax.ShapeDtypeStruct((3, 80, 128), jnp.int32),
        jax.ShapeDtypeStruct((80, 128), jnp.float32),
        jax.ShapeDtypeStruct((392, 128), jnp.int32),
        jax.ShapeDtypeStruct((1, 1), jnp.float32),
    ),
)


@functools.cache
def _get_sc_loglik():
    mesh = plsc.VectorSubcoreMesh(core_axis_name="c", subcore_axis_name="s")
    cp = pltpu.CompilerParams()
    if "needs_layout_passes" in pltpu.CompilerParams.__dataclass_fields__:
        cp = dataclasses.replace(cp, needs_layout_passes=False)
    return pl.kernel(
        _sc_loglik_body,
        out_type=jax.ShapeDtypeStruct((_NW, 16), jnp.float32),
        mesh=mesh,
        scratch_types=[
            pltpu.VMEM((3, _IP), jnp.int32),
            pltpu.VMEM((_IP,), jnp.float32),
            pltpu.VMEM((_TPH,), jnp.int32),
            pltpu.VMEM((16,), jnp.float32),
        ],
        compiler_params=cp,
    )


def _sc_loglik_body(c_hbm, a_hbm, tp_hbm, item_hbm, person_hbm, resp_hbm,
                    out_hbm, c_v, a_v, tp_v, acc_v):
    pltpu.sync_copy(c_hbm, c_v)
    pltpu.sync_copy(a_hbm, a_v)
    pltpu.sync_copy(tp_hbm, tp_v)
    acc_v[...] = jnp.zeros((16,), jnp.float32)

    def chunk_body(item_vmem, person_vmem, resp_vmem):
        @pl.loop(0, _CH, step=16)
        def _(r0):
            item = item_vmem[pl.ds(r0, 16)]
            person = person_vmem[pl.ds(r0, 16)]
            resp = resp_vmem[pl.ds(r0, 16)]

            g = resp - 1
            cw1 = plsc.load_gather(c_v, [g >> 1, item])
            cw2 = plsc.load_gather(c_v, [(g + 1) >> 1, item])
            a = plsc.load_gather(a_v, [item])
            in_lo = person < _TPH
            tw = plsc.load_gather(
                tp_v, [jnp.where(in_lo, person, person - _TPH)])
            t_bits = jnp.where(in_lo, tw << 16, tw & jnp.int32(-65536))
            t = plsc.bitcast(t_bits, jnp.float32)
            godd = (g & 1) == 1
            ab_lo = plsc.bitcast(
                jnp.where(godd, cw1 & jnp.int32(-65536), cw1 << 16),
                jnp.float32)
            ab_hi = plsc.bitcast(
                jnp.where(godd, cw2 << 16, cw2 & jnp.int32(-65536)),
                jnp.float32)

            at = a * t
            x1 = jnp.minimum(jnp.maximum(at - ab_lo, -_CLAMP), _CLAMP)
            x2 = jnp.minimum(jnp.maximum(at - ab_hi, -_CLAMP), _CLAMP)
            u = jnp.exp(-x1)
            v = jnp.exp(-x2)
            p = (v - u) / ((1.0 + u) * (1.0 + v))

            bits = plsc.bitcast(p, jnp.int32)
            e = (bits >> 23) - 127
            m = plsc.bitcast((bits & 0x007FFFFF) | 0x3F800000, jnp.float32)
            big = m > 1.4142135
            m = jnp.where(big, 0.5 * m, m)
            ef = (e + big.astype(jnp.int32)).astype(jnp.float32)
            r = (m - 1.0) / (m + 1.0)
            s = r * r
            lm = 2.0 * r * (1.0 + s * (1.0 / 3.0 + s * (0.2 + s * (1.0 / 7.0))))
            acc_v[...] = acc_v[...] + (ef * _LN2 + lm)

    pltpu.emit_pipeline(
        chunk_body,
        grid=(_NCHUNKS,),
        in_specs=[pl.BlockSpec((_CH,), lambda i: (i,)),
                  pl.BlockSpec((_CH,), lambda i: (i,)),
                  pl.BlockSpec((_CH,), lambda i: (i,))],
        core_axis_name=("c", "s"),
        dimension_semantics=(pltpu.PARALLEL,),
    )(item_hbm, person_hbm, resp_hbm)

    wid = lax.axis_index("c") * 16 + lax.axis_index("s")
    pltpu.sync_copy(acc_v, out_hbm.at[wid])


def kernel(indices, a_, b_base_, b_diff_, t):
    n_resp = indices.shape[0]
    item_v = indices[:, 0]
    person_v = indices[:, 1]
    resp_v = indices[:, 2]

    a_in = jnp.pad(a_, (0, _IP - _N_ITEMS)).reshape(80, 128)
    bb_in = jnp.pad(b_base_[:, 0], (0, _IP - _N_ITEMS)).reshape(80, 128)
    bd_in = jnp.pad(b_diff_, ((0, _IP - _N_ITEMS), (0, 0))).T.reshape(3, 80, 128)
    t_in = jnp.pad(t, (0, _PP - _N_PERSONS)).reshape(784, 128)

    c_tab, a_tab, tp_tab, prior = _prep(a_in, bb_in, bd_in, t_in)
    parts = _get_sc_loglik()(
        c_tab.reshape(3, _IP),
        a_tab.reshape(_IP),
        tp_tab.reshape(_TPH),
        item_v,
        person_v,
        resp_v,
    )
    prior_scale = n_resp / 1000000.0
    return -(jnp.sum(parts) + prior[0, 0] * prior_scale)


# speedup vs baseline: 25.4227x; 1.5972x over previous
import dataclasses
import functools

import jax
import jax.numpy as jnp
from jax import lax
from jax.experimental import pallas as pl
from jax.experimental.pallas import tpu as pltpu
from jax.experimental.pallas import tpu_sc as plsc

_N_ITEMS = 10000
_N_PERSONS = 100000
_N_GRADES = 5
_IP = 10240
_PP = 100352
_TPH = _PP // 2
_NW = 32
_N_RESP = 1000000
_CH = 800
_NCHUNKS = _N_RESP // _CH
_BIG = 30000.0
_CLAMP = 30.0
_LOG2PI = 1.8378770664093453
_LN2 = 0.6931471805599453


def _bf16_bits(x):
    u = lax.bitcast_convert_type(x, jnp.int32)
    return ((u + 0x7FFF + ((u >> 16) & 1)) >> 16) & 0xFFFF


def _pack_pair(lo, hi):
    return (_bf16_bits(hi) << 16) | _bf16_bits(lo)


def _prep_body(a_ref, bb_ref, bd_ref, t_ref, c_ref, a_out_ref, tpk_ref,
               prior_ref):
    rows = lax.broadcasted_iota(jnp.int32, (80, 128), 0)
    cols = lax.broadcasted_iota(jnp.int32, (80, 128), 1)
    item_idx = rows * 128 + cols
    valid_item = item_idx < _N_ITEMS

    a_raw = a_ref[...]
    a = jnp.log(1.0 + jnp.exp(a_raw))
    g1 = jnp.log(1.0 + jnp.exp(bd_ref[0]))
    g2 = jnp.log(1.0 + jnp.exp(bd_ref[1]))
    g3 = jnp.log(1.0 + jnp.exp(bd_ref[2]))
    b1 = bb_ref[...]
    b2 = b1 + g1
    b3 = b2 + g2
    b4 = b3 + g3

    def npdf_sum(x, mask):
        return jnp.sum(jnp.where(mask, -0.5 * x * x - 0.5 * _LOG2PI, 0.0))

    t_all = t_ref[...]
    prows = lax.broadcasted_iota(jnp.int32, (784, 128), 0)
    pcols = lax.broadcasted_iota(jnp.int32, (784, 128), 1)
    valid_person = (prows * 128 + pcols) < _N_PERSONS
    prior = (npdf_sum(a, valid_item)
             + npdf_sum(b1, valid_item) + npdf_sum(b2, valid_item)
             + npdf_sum(b3, valid_item) + npdf_sum(b4, valid_item)
             + npdf_sum(t_all, valid_person))
    prior_ref[...] = jnp.full((1, 1), prior, jnp.float32)

    a_out_ref[...] = jnp.where(valid_item, a, 1.0)
    ab_raw = (a * -1000.0, a * b1, a * b2, a * b3, a * b4, a * 1000.0)
    pad_val = (-_BIG, _BIG, _BIG, _BIG, _BIG, _BIG)
    ab = tuple(jnp.where(valid_item, ab_raw[s], pad_val[s]) for s in range(6))
    for w in range(3):
        c_ref[w] = _pack_pair(ab[2 * w], ab[2 * w + 1])
    tpk_ref[...] = _pack_pair(t_ref[0:392], t_ref[392:784])


_DB = 8192


def _deint_body(idx_ref, item_ref, person_ref, resp_ref):
    x = idx_ref[...]
    lane = lax.broadcasted_iota(jnp.int32, (_DB, 3), 1)
    item_ref[...] = jnp.sum(jnp.where(lane == 0, x, 0), axis=1)
    person_ref[...] = jnp.sum(jnp.where(lane == 1, x, 0), axis=1)
    resp_ref[...] = jnp.sum(jnp.where(lane == 2, x, 0), axis=1)


_deint = pl.pallas_call(
    _deint_body,
    grid=(pl.cdiv(_N_RESP, _DB),),
    in_specs=[pl.BlockSpec((_DB, 3), lambda i: (i, 0))],
    out_specs=(
        pl.BlockSpec((_DB,), lambda i: (i,)),
        pl.BlockSpec((_DB,), lambda i: (i,)),
        pl.BlockSpec((_DB,), lambda i: (i,)),
    ),
    out_shape=(
        jax.ShapeDtypeStruct((_N_RESP,), jnp.int32),
        jax.ShapeDtypeStruct((_N_RESP,), jnp.int32),
        jax.ShapeDtypeStruct((_N_RESP,), jnp.int32),
    ),
)


_prep = pl.pallas_call(
    _prep_body,
    out_shape=(
        jax.ShapeDtypeStruct((3, 80, 128), jnp.int32),
        jax.ShapeDtypeStruct((80, 128), jnp.float32),
        jax.ShapeDtypeStruct((392, 128), jnp.int32),
        jax.ShapeDtypeStruct((1, 1), jnp.float32),
    ),
)


@functools.cache
def _get_sc_loglik():
    mesh = plsc.VectorSubcoreMesh(core_axis_name="c", subcore_axis_name="s")
    cp = pltpu.CompilerParams()
    if "needs_layout_passes" in pltpu.CompilerParams.__dataclass_fields__:
        cp = dataclasses.replace(cp, needs_layout_passes=False)
    return pl.kernel(
        _sc_loglik_body,
        out_type=jax.ShapeDtypeStruct((_NW, 16), jnp.float32),
        mesh=mesh,
        scratch_types=[
            pltpu.VMEM((3, _IP), jnp.int32),
            pltpu.VMEM((_IP,), jnp.float32),
            pltpu.VMEM((_TPH,), jnp.int32),
            pltpu.VMEM((16,), jnp.float32),
        ],
        compiler_params=cp,
    )


def _sc_loglik_body(c_hbm, a_hbm, tp_hbm, item_hbm, person_hbm, resp_hbm,
                    out_hbm, c_v, a_v, tp_v, acc_v):
    pltpu.sync_copy(c_hbm, c_v)
    pltpu.sync_copy(a_hbm, a_v)
    pltpu.sync_copy(tp_hbm, tp_v)
    acc_v[...] = jnp.zeros((16,), jnp.float32)

    def _mlog(x):
        bits = plsc.bitcast(x, jnp.int32)
        ef = ((bits >> 23) - 127).astype(jnp.float32)
        w = plsc.bitcast((bits & 0x007FFFFF) | 0x3F800000, jnp.float32) - 1.0
        poly = jnp.float32(-0.00846516958429516)
        for c in (0.04365465633349474, -0.10679236386265466,
                  0.17659330219989444, -0.24453166236624596,
                  0.3326520724645769, -0.49996351722683513,
                  0.9999995170630975):
            poly = poly * w + c
        return ef * _LN2 + w * poly

    def chunk_body(item_vmem, person_vmem, resp_vmem):
        @pl.loop(0, _CH, step=16, init_carry=jnp.zeros((16,), jnp.float32),
                 unroll=4)
        def loop(r0, acc):
            item = item_vmem[pl.ds(r0, 16)]
            person = person_vmem[pl.ds(r0, 16)]
            resp = resp_vmem[pl.ds(r0, 16)]

            g = resp - 1
            godd_i = g & 1
            w1 = g >> 1
            cw1 = plsc.load_gather(c_v, [w1, item])
            cw2 = plsc.load_gather(c_v, [w1 + godd_i, item])
            a = plsc.load_gather(a_v, [item])
            in_lo = person < _TPH
            tw = plsc.load_gather(
                tp_v, [jnp.where(in_lo, person, person - _TPH)])
            t_bits = jnp.where(in_lo, tw << 16, tw & jnp.int32(-65536))
            t = plsc.bitcast(t_bits, jnp.float32)
            godd = godd_i == 1
            ab_lo = plsc.bitcast(
                jnp.where(godd, cw1 & jnp.int32(-65536), cw1 << 16),
                jnp.float32)
            ab_hi = plsc.bitcast(
                jnp.where(godd, cw2 << 16, cw2 & jnp.int32(-65536)),
                jnp.float32)

            at = a * t
            x1 = jnp.maximum(at - ab_lo, -_CLAMP)
            x2 = jnp.maximum(at - ab_hi, -_CLAMP)
            u = jnp.exp(-x1)
            v = jnp.exp(-x2)
            num = v - u
            den = (1.0 + u) * (1.0 + v)
            return acc + (_mlog(num) - _mlog(den))

        acc_v[...] = acc_v[...] + loop

    pltpu.emit_pipeline(
        chunk_body,
        grid=(_NCHUNKS,),
        in_specs=[pl.BlockSpec((_CH,), lambda i: (i,)),
                  pl.BlockSpec((_CH,), lambda i: (i,)),
                  pl.BlockSpec((_CH,), lambda i: (i,))],
        core_axis_name=("c", "s"),
        dimension_semantics=(pltpu.PARALLEL,),
    )(item_hbm, person_hbm, resp_hbm)

    wid = lax.axis_index("c") * 16 + lax.axis_index("s")
    pltpu.sync_copy(acc_v, out_hbm.at[wid])


def kernel(indices, a_, b_base_, b_diff_, t):
    n_resp = indices.shape[0]
    item_v = indices[:, 0]
    person_v = indices[:, 1]
    resp_v = indices[:, 2]

    a_in = jnp.pad(a_, (0, _IP - _N_ITEMS)).reshape(80, 128)
    bb_in = jnp.pad(b_base_[:, 0], (0, _IP - _N_ITEMS)).reshape(80, 128)
    bd_in = jnp.pad(b_diff_, ((0, _IP - _N_ITEMS), (0, 0))).T.reshape(3, 80, 128)
    t_in = jnp.pad(t, (0, _PP - _N_PERSONS)).reshape(784, 128)

    c_tab, a_tab, tp_tab, prior = _prep(a_in, bb_in, bd_in, t_in)
    parts = _get_sc_loglik()(
        c_tab.reshape(3, _IP),
        a_tab.reshape(_IP),
        tp_tab.reshape(_TPH),
        item_v,
        person_v,
        resp_v,
    )
    prior_scale = n_resp / 1000000.0
    return -(jnp.sum(parts) + prior[0, 0] * prior_scale)
